# baseline (speedup 1.0000x reference)
import jax
import jax.numpy as jnp
from jax import lax
from jax.experimental import pallas as pl
from jax.experimental.pallas import tpu as pltpu
from jax.experimental.pallas import tpu_sc as plsc

N_ATOMS = 4194304
N_SYSTEMS = 16384
N_SPECIES = 10
NC = 2
NS = 16
L = 16
NW = NC * NS
APW = N_ATOMS // NW
CHUNK = 8192
NCHUNK = APW // CHUNK


def _sc_body(species_hbm, sys_hbm, table_hbm, out_hbm,
             table_v, acc_v, sp_v, sy_v, sem0, sem1):
    c = lax.axis_index("c")
    s = lax.axis_index("s")
    wid = s * NC + c
    base = wid * APW
    sems = (sem0, sem1)

    zeros = jnp.zeros((L,), jnp.float32)

    def _issue(ci, b):
        off = base + ci * CHUNK
        pltpu.async_copy(species_hbm.at[pl.ds(off, CHUNK)], sp_v.at[b],
                         sems[b])
        pltpu.async_copy(sys_hbm.at[pl.ds(off, CHUNK)], sy_v.at[b], sems[b])

    def _wait(ci, b):
        off = base + ci * CHUNK
        pltpu.make_async_copy(species_hbm.at[pl.ds(off, CHUNK)], sp_v.at[b],
                              sems[b]).wait()
        pltpu.make_async_copy(sys_hbm.at[pl.ds(off, CHUNK)], sy_v.at[b],
                              sems[b]).wait()

    lane = lax.iota(jnp.int32, L)
    is_last = lane == (L - 1)
    shift1 = jnp.minimum(lane + 1, L - 1)

    def _compute(b, tv):
        @plsc.parallel_loop(0, CHUNK, step=L, unroll=8)
        def _atoms(i):
            sp = sp_v[b, pl.ds(i, L)]
            sy = sy_v[b, pl.ds(i, L)]
            sy2 = sy.at[shift1].get(mode="promise_in_bounds")
            e = tv.at[sp].get(mode="promise_in_bounds")
            c = plsc.cumsum(e)
            mb = sy != sy2
            me = jnp.logical_or(mb, is_last)
            plsc.addupdate_scatter(acc_v, [sy], c, mask=me)
            plsc.addupdate_scatter(acc_v, [sy2], -c, mask=mb)

    _issue(0, 0)
    _issue(1, 1)
    pltpu.sync_copy(table_hbm, table_v.at[pl.ds(0, N_SPECIES)])
    tv = table_v[...]

    @plsc.parallel_loop(0, N_SYSTEMS, step=L, unroll=8)
    def _zero(i):
        acc_v[pl.ds(i, L)] = zeros

    def pair_body(p, carry):
        ci0 = p * 2
        for b in range(2):
            ci = ci0 + b
            _wait(ci, b)
            _compute(b, tv)

            @pl.when(ci + 2 < NCHUNK)
            def _():
                _issue(ci + 2, b)
        return carry

    lax.fori_loop(0, NCHUNK // 2, pair_body, None)
    pltpu.sync_copy(acc_v, out_hbm.at[wid])


_sc_kernel = pl.kernel(
    _sc_body,
    out_type=jax.ShapeDtypeStruct((NW, N_SYSTEMS), jnp.float32),
    mesh=plsc.VectorSubcoreMesh(core_axis_name="c", subcore_axis_name="s"),
    compiler_params=pltpu.CompilerParams(needs_layout_passes=False),
    scratch_types=[
        pltpu.VMEM((L,), jnp.float32),
        pltpu.VMEM((N_SYSTEMS,), jnp.float32),
        pltpu.VMEM((2, CHUNK), jnp.int32),
        pltpu.VMEM((2, CHUNK), jnp.int32),
        pltpu.SemaphoreType.DMA,
        pltpu.SemaphoreType.DMA,
    ],
)


def _reduce_body(p_ref, o_ref):
    o_ref[...] = jnp.sum(p_ref[...], axis=0, keepdims=True)


_reduce = pl.pallas_call(
    _reduce_body,
    out_shape=jax.ShapeDtypeStruct((1, N_SYSTEMS), jnp.float32),
)


def kernel(species, system_ids, n_systems, energy_table):
    partials = _sc_kernel(species, system_ids, energy_table)
    energy = _reduce(partials)
    return jnp.reshape(energy, (N_SYSTEMS, 1))


# speedup vs baseline: 1.0008x; 1.0008x over previous
import jax
import jax.numpy as jnp
from jax import lax
from jax.experimental import pallas as pl
from jax.experimental.pallas import tpu as pltpu
from jax.experimental.pallas import tpu_sc as plsc

N_ATOMS = 4194304
N_SYSTEMS = 16384
N_SPECIES = 10
NC = 2
NS = 16
L = 16
NW = NC * NS
APW = N_ATOMS // NW
CHUNK = 16384
NCHUNK = APW // CHUNK


def _sc_body(species_hbm, sys_hbm, table_hbm, out_hbm,
             table_v, acc_v, sp_v, sy_v, sem0, sem1):
    c = lax.axis_index("c")
    s = lax.axis_index("s")
    wid = s * NC + c
    base = wid * APW
    sems = (sem0, sem1)

    zeros = jnp.zeros((L,), jnp.float32)

    def _issue(ci, b):
        off = base + ci * CHUNK
        pltpu.async_copy(species_hbm.at[pl.ds(off, CHUNK)], sp_v.at[b],
                         sems[b])
        pltpu.async_copy(sys_hbm.at[pl.ds(off, CHUNK)], sy_v.at[b], sems[b])

    def _wait(ci, b):
        off = base + ci * CHUNK
        pltpu.make_async_copy(species_hbm.at[pl.ds(off, CHUNK)], sp_v.at[b],
                              sems[b]).wait()
        pltpu.make_async_copy(sys_hbm.at[pl.ds(off, CHUNK)], sy_v.at[b],
                              sems[b]).wait()

    lane = lax.iota(jnp.int32, L)
    is_last = lane == (L - 1)
    shift1 = jnp.minimum(lane + 1, L - 1)

    def _compute(b, tv):
        @plsc.parallel_loop(0, CHUNK, step=L, unroll=8)
        def _atoms(i):
            sp = sp_v[b, pl.ds(i, L)]
            sy = sy_v[b, pl.ds(i, L)]
            sy2 = sy.at[shift1].get(mode="promise_in_bounds")
            e = tv.at[sp].get(mode="promise_in_bounds")
            c = plsc.cumsum(e)
            mb = sy != sy2
            me = jnp.logical_or(mb, is_last)
            plsc.addupdate_scatter(acc_v, [sy], c, mask=me)
            plsc.addupdate_scatter(acc_v, [sy2], -c, mask=mb)

    _issue(0, 0)
    _issue(1, 1)
    pltpu.sync_copy(table_hbm, table_v.at[pl.ds(0, N_SPECIES)])
    tv = table_v[...]

    @plsc.parallel_loop(0, N_SYSTEMS, step=L, unroll=8)
    def _zero(i):
        acc_v[pl.ds(i, L)] = zeros

    def pair_body(p, carry):
        ci0 = p * 2
        for b in range(2):
            ci = ci0 + b
            _wait(ci, b)
            _compute(b, tv)

            @pl.when(ci + 2 < NCHUNK)
            def _():
                _issue(ci + 2, b)
        return carry

    lax.fori_loop(0, NCHUNK // 2, pair_body, None)
    pltpu.sync_copy(acc_v, out_hbm.at[wid])


_sc_kernel = pl.kernel(
    _sc_body,
    out_type=jax.ShapeDtypeStruct((NW, N_SYSTEMS), jnp.float32),
    mesh=plsc.VectorSubcoreMesh(core_axis_name="c", subcore_axis_name="s"),
    compiler_params=pltpu.CompilerParams(needs_layout_passes=False),
    scratch_types=[
        pltpu.VMEM((L,), jnp.float32),
        pltpu.VMEM((N_SYSTEMS,), jnp.float32),
        pltpu.VMEM((2, CHUNK), jnp.int32),
        pltpu.VMEM((2, CHUNK), jnp.int32),
        pltpu.SemaphoreType.DMA,
        pltpu.SemaphoreType.DMA,
    ],
)


def _reduce_body(p_ref, o_ref):
    o_ref[...] = jnp.sum(p_ref[...], axis=0, keepdims=True)


_reduce = pl.pallas_call(
    _reduce_body,
    out_shape=jax.ShapeDtypeStruct((1, N_SYSTEMS), jnp.float32),
)


def kernel(species, system_ids, n_systems, energy_table):
    partials = _sc_kernel(species, system_ids, energy_table)
    energy = _reduce(partials)
    return jnp.reshape(energy, (N_SYSTEMS, 1))
